# Initial kernel scaffold; baseline (speedup 1.0000x reference)
import jax
import jax.numpy as jnp
from jax.experimental import pallas as pl


def kernel(x, Wr, W1, W2, W3):
    raise NotImplementedError("write your pallas kernel here")


# speedup vs baseline: 1.0265x; 1.0265x over previous
import jax
import jax.numpy as jnp
from jax.experimental import pallas as pl
from jax.experimental.pallas import tpu as pltpu

K = 2
B_T = 128
F_TILE = 1024


def _ffn_kernel(be_ref, x_ref, w1_ref, w3_ref, w2_ref, out_ref, acc_ref):
    f = pl.program_id(0)
    j = pl.program_id(1)
    nf = pl.num_programs(0)
    x = x_ref[...]
    h1 = jnp.dot(x, w1_ref[0], preferred_element_type=jnp.float32)
    h3 = jnp.dot(x, w3_ref[0], preferred_element_type=jnp.float32)
    act = (h1 * jax.nn.sigmoid(h1) * h3).astype(jnp.bfloat16)
    part = jnp.dot(act, w2_ref[0], preferred_element_type=jnp.float32)
    row = pl.multiple_of(j * B_T, B_T)

    @pl.when(f == 0)
    def _():
        acc_ref[pl.ds(row, B_T), :] = part

    @pl.when(f != 0)
    def _():
        acc_ref[pl.ds(row, B_T), :] += part

    @pl.when(f == nf - 1)
    def _():
        out_ref[...] = acc_ref[pl.ds(row, B_T), :]


def kernel(x, Wr, W1, W2, W3):
    b, s, h = x.shape
    T = b * s
    E = Wr.shape[1]
    F = W1.shape[2]
    A = T * K
    NF = F // F_TILE
    NB = A // B_T + E
    P = NB * B_T

    xf = x.reshape(T, h)

    logits = xf @ Wr
    top_vals, top_idx = jax.lax.top_k(logits, K)
    rw = jax.nn.softmax(top_vals, axis=-1)

    ef = top_idx.reshape(A)
    onehot = (ef[:, None] == jnp.arange(E, dtype=ef.dtype)[None, :]).astype(jnp.int32)
    counts = onehot.sum(0)
    csum = jnp.cumsum(counts)
    offsets = csum - counts
    rank = jnp.take_along_axis(jnp.cumsum(onehot, axis=0), ef[:, None], axis=1)[:, 0] - 1

    nblk = (counts + B_T - 1) // B_T
    blk_incl = jnp.cumsum(nblk)
    blk_excl = blk_incl - nblk
    block_e = jnp.clip(
        jnp.searchsorted(blk_incl, jnp.arange(NB, dtype=jnp.int32), side="right"),
        0, E - 1).astype(jnp.int32)

    p_a = (blk_excl[ef] + rank // B_T) * B_T + rank % B_T

    tok = jnp.arange(A, dtype=jnp.int32) // K
    tok_pad = jnp.zeros((P,), jnp.int32).at[p_a].set(tok)
    xs_pad = xf[tok_pad].astype(jnp.bfloat16)

    W1b = W1.astype(jnp.bfloat16)
    W3b = W3.astype(jnp.bfloat16)
    W2b = W2.astype(jnp.bfloat16)

    grid_spec = pltpu.PrefetchScalarGridSpec(
        num_scalar_prefetch=1,
        grid=(NF, NB),
        in_specs=[
            pl.BlockSpec((B_T, h), lambda f, j, be: (j, 0)),
            pl.BlockSpec((1, h, F_TILE), lambda f, j, be: (be[j], 0, f)),
            pl.BlockSpec((1, h, F_TILE), lambda f, j, be: (be[j], 0, f)),
            pl.BlockSpec((1, F_TILE, h), lambda f, j, be: (be[j], f, 0)),
        ],
        out_specs=pl.BlockSpec(
            (B_T, h), lambda f, j, be: (jnp.where(f == NF - 1, j, 0), 0)),
        scratch_shapes=[pltpu.VMEM((P, h), jnp.float32)],
    )
    Y = pl.pallas_call(
        _ffn_kernel,
        grid_spec=grid_spec,
        out_shape=jax.ShapeDtypeStruct((P, h), jnp.float32),
        compiler_params=pltpu.CompilerParams(
            dimension_semantics=("arbitrary", "arbitrary"),
            vmem_limit_bytes=56 * 1024 * 1024,
        ),
    )(block_e, xs_pad, W1b, W3b, W2b)

    sel = Y[p_a].reshape(T, K, h)
    final = (sel * rw[:, :, None]).sum(1).reshape(b, s, h)
    metrics = jnp.sqrt((sel * sel).sum(-1)).reshape(b, s, K)
    return (final,
            rw.reshape(b, s, K),
            top_idx.reshape(b, s, K),
            metrics)


# speedup vs baseline: 1.2929x; 1.2595x over previous
import jax
import jax.numpy as jnp
from jax.experimental import pallas as pl
from jax.experimental.pallas import tpu as pltpu

K = 2
B_T = 128
F_TILE = 1024


def _ffn_kernel(be_ref, x_ref, w1_ref, w3_ref, w2_ref, out_ref, acc_ref):
    f = pl.program_id(0)
    j = pl.program_id(1)
    nf = pl.num_programs(0)
    x = x_ref[...]
    h1 = jnp.dot(x, w1_ref[0], preferred_element_type=jnp.float32)
    h3 = jnp.dot(x, w3_ref[0], preferred_element_type=jnp.float32)
    act = h1 * jax.nn.sigmoid(h1) * h3
    part = jnp.dot(act, w2_ref[0], preferred_element_type=jnp.float32)
    row = pl.multiple_of(j * B_T, B_T)

    @pl.when(f == 0)
    def _():
        acc_ref[pl.ds(row, B_T), :] = part

    @pl.when(f != 0)
    def _():
        acc_ref[pl.ds(row, B_T), :] += part

    @pl.when(f == nf - 1)
    def _():
        out_ref[...] = acc_ref[pl.ds(row, B_T), :]


def kernel(x, Wr, W1, W2, W3):
    b, s, h = x.shape
    T = b * s
    E = Wr.shape[1]
    F = W1.shape[2]
    A = T * K
    NF = F // F_TILE
    NB = A // B_T + E
    P = NB * B_T

    xf = x.reshape(T, h)

    logits = xf @ Wr
    top_vals, top_idx = jax.lax.top_k(logits, K)
    rw = jax.nn.softmax(top_vals, axis=-1)

    ef = top_idx.reshape(A)
    onehot = (ef[:, None] == jnp.arange(E, dtype=ef.dtype)[None, :]).astype(jnp.int32)
    counts = onehot.sum(0)
    csum = jnp.cumsum(counts)
    offsets = csum - counts
    rank = jnp.take_along_axis(jnp.cumsum(onehot, axis=0), ef[:, None], axis=1)[:, 0] - 1

    nblk = (counts + B_T - 1) // B_T
    blk_incl = jnp.cumsum(nblk)
    blk_excl = blk_incl - nblk
    block_e = jnp.clip(
        jnp.searchsorted(blk_incl, jnp.arange(NB, dtype=jnp.int32), side="right"),
        0, E - 1).astype(jnp.int32)

    p_a = (blk_excl[ef] + rank // B_T) * B_T + rank % B_T

    tok = jnp.arange(A, dtype=jnp.int32) // K
    tok_pad = jnp.zeros((P,), jnp.int32).at[p_a].set(tok)
    xs_pad = xf[tok_pad]

    grid_spec = pltpu.PrefetchScalarGridSpec(
        num_scalar_prefetch=1,
        grid=(NF, NB),
        in_specs=[
            pl.BlockSpec((B_T, h), lambda f, j, be: (j, 0)),
            pl.BlockSpec((1, h, F_TILE), lambda f, j, be: (be[j], 0, f)),
            pl.BlockSpec((1, h, F_TILE), lambda f, j, be: (be[j], 0, f)),
            pl.BlockSpec((1, F_TILE, h), lambda f, j, be: (be[j], f, 0)),
        ],
        out_specs=pl.BlockSpec(
            (B_T, h), lambda f, j, be: (jnp.where(f == NF - 1, j, 0), 0)),
        scratch_shapes=[pltpu.VMEM((P, h), jnp.float32)],
    )
    Y = pl.pallas_call(
        _ffn_kernel,
        grid_spec=grid_spec,
        out_shape=jax.ShapeDtypeStruct((P, h), jnp.float32),
        compiler_params=pltpu.CompilerParams(
            dimension_semantics=("arbitrary", "arbitrary"),
            vmem_limit_bytes=56 * 1024 * 1024,
        ),
    )(block_e, xs_pad, W1, W3, W2)

    sel = Y[p_a].reshape(T, K, h)
    final = (sel * rw[:, :, None]).sum(1).reshape(b, s, h)
    metrics = jnp.sqrt((sel * sel).sum(-1)).reshape(b, s, K)
    return (final,
            rw.reshape(b, s, K),
            top_idx.reshape(b, s, K),
            metrics)


# speedup vs baseline: 1.3864x; 1.0723x over previous
import jax
import jax.numpy as jnp
from jax.experimental import pallas as pl
from jax.experimental.pallas import tpu as pltpu

K = 2
B_T = 256
F_TILE = 1024


def _ffn_kernel(be_ref, x_ref, w1_ref, w3_ref, w2_ref, out_ref, acc_ref):
    f = pl.program_id(0)
    j = pl.program_id(1)
    nf = pl.num_programs(0)
    x = x_ref[...]
    h1 = jnp.dot(x, w1_ref[0], preferred_element_type=jnp.float32)
    h3 = jnp.dot(x, w3_ref[0], preferred_element_type=jnp.float32)
    act = h1 * jax.nn.sigmoid(h1) * h3
    part = jnp.dot(act, w2_ref[0], preferred_element_type=jnp.float32)
    row = pl.multiple_of(j * B_T, B_T)

    @pl.when(f == 0)
    def _():
        acc_ref[pl.ds(row, B_T), :] = part

    @pl.when((f != 0) & (f != nf - 1))
    def _():
        acc_ref[pl.ds(row, B_T), :] += part

    @pl.when(f == nf - 1)
    def _():
        out_ref[...] = acc_ref[pl.ds(row, B_T), :] + part


def kernel(x, Wr, W1, W2, W3):
    b, s, h = x.shape
    T = b * s
    E = Wr.shape[1]
    F = W1.shape[2]
    A = T * K
    NF = F // F_TILE
    NB = A // B_T + E
    P = NB * B_T

    xf = x.reshape(T, h)

    logits = xf @ Wr
    top_vals, top_idx = jax.lax.top_k(logits, K)
    rw = jax.nn.softmax(top_vals, axis=-1)

    ef = top_idx.reshape(A)
    onehot = (ef[:, None] == jnp.arange(E, dtype=ef.dtype)[None, :]).astype(jnp.int32)
    counts = onehot.sum(0)
    csum = jnp.cumsum(counts)
    offsets = csum - counts
    rank = jnp.take_along_axis(jnp.cumsum(onehot, axis=0), ef[:, None], axis=1)[:, 0] - 1

    nblk = (counts + B_T - 1) // B_T
    blk_incl = jnp.cumsum(nblk)
    blk_excl = blk_incl - nblk
    block_e = jnp.clip(
        jnp.searchsorted(blk_incl, jnp.arange(NB, dtype=jnp.int32), side="right"),
        0, E - 1).astype(jnp.int32)

    p_a = (blk_excl[ef] + rank // B_T) * B_T + rank % B_T

    tok = jnp.arange(A, dtype=jnp.int32) // K
    tok_pad = jnp.zeros((P,), jnp.int32).at[p_a].set(tok)
    xs_pad = xf[tok_pad]

    grid_spec = pltpu.PrefetchScalarGridSpec(
        num_scalar_prefetch=1,
        grid=(NF, NB),
        in_specs=[
            pl.BlockSpec((B_T, h), lambda f, j, be: (j, 0)),
            pl.BlockSpec((1, h, F_TILE), lambda f, j, be: (be[j], 0, f)),
            pl.BlockSpec((1, h, F_TILE), lambda f, j, be: (be[j], 0, f)),
            pl.BlockSpec((1, F_TILE, h), lambda f, j, be: (be[j], f, 0)),
        ],
        out_specs=pl.BlockSpec(
            (B_T, h), lambda f, j, be: (jnp.where(f == NF - 1, j, 0), 0)),
        scratch_shapes=[pltpu.VMEM((P, h), jnp.float32)],
    )
    Y = pl.pallas_call(
        _ffn_kernel,
        grid_spec=grid_spec,
        out_shape=jax.ShapeDtypeStruct((P, h), jnp.float32),
        compiler_params=pltpu.CompilerParams(
            dimension_semantics=("arbitrary", "arbitrary"),
            vmem_limit_bytes=56 * 1024 * 1024,
        ),
    )(block_e, xs_pad, W1, W3, W2)

    sel = Y[p_a].reshape(T, K, h)
    final = (sel * rw[:, :, None]).sum(1).reshape(b, s, h)
    metrics = jnp.sqrt((sel * sel).sum(-1)).reshape(b, s, K)
    return (final,
            rw.reshape(b, s, K),
            top_idx.reshape(b, s, K),
            metrics)
